# baseline (speedup 1.0000x reference)
import functools

import jax
import jax.numpy as jnp
import numpy as np
from jax import lax
from jax.experimental import pallas as pl
from jax.experimental.pallas import tpu as pltpu
from jax.experimental.pallas import tpu_sc as plsc

BATCH = 16384
WIDTH = 128
NROWS_PAD = 1320
ROWS = BATCH * 4
NC, NS = 2, 16
NW = NC * NS
RPW = ROWS // NW
CH = 128
NCH = RPW // CH


def _static_tables():
    sig = np.zeros((4, 9), dtype=np.int64)
    for p in range(9):
        x, y = divmod(p, 3)
        sig[0, p] = p
        sig[1, p] = 3 * (2 - y) + x
        sig[2, p] = 8 - p
        sig[3, p] = 3 * y + (2 - x)
    inv = np.zeros_like(sig)
    for j in range(4):
        inv[j, sig[j]] = np.arange(9)
    q_src = [0, 2, 3, 1]

    m = np.zeros((NROWS_PAD, 46), np.float32)
    for j in range(4):
        base = 324 * j
        for ia in range(81):
            v = ia
            for t in range(4):
                d = v % 3
                v //= 3
                m[base + ia, 3 * inv[j][t] + d] += 1
        for ib in range(243):
            v = ib
            for t in range(5):
                d = v % 3
                v //= 3
                m[base + 81 + ib, 3 * inv[j][4 + t] + d] += 1
    for c in range(19):
        m[1296 + c, 27 + c] = 1

    widx = np.zeros((36, 8), np.float32)
    for j in range(4):
        for t in range(4):
            widx[9 * q_src[j] + t, j] = 3 ** t
        for t in range(5):
            widx[9 * q_src[j] + 4 + t, 4 + j] = 3 ** t
    return m, widx


_M, _WIDX = _static_tables()


def _table_body(m_ref, w_ref, t_ref):
    t_ref[...] = jnp.dot(m_ref[...], w_ref[...],
                         preferred_element_type=jnp.float32)


def _idx_body(boards_ref, widx_ref, ia_ref, ib_ref, ic_ref):
    bf = boards_ref[...].astype(jnp.float32)
    prods = jnp.dot(bf, widx_ref[...], preferred_element_type=jnp.float32)
    prods = prods.astype(jnp.int32)
    joff = 324 * lax.broadcasted_iota(jnp.int32, (1, 4), 1)
    ia_ref[...] = prods[:, :4] + joff
    ib_ref[...] = prods[:, 4:8] + joff + 81
    cnt = jnp.sum((boards_ref[...] != 0).astype(jnp.int32), axis=1,
                  keepdims=True)
    ic = jnp.where(cnt < 19, 1296 + cnt, 1315)
    ic_ref[...] = jnp.broadcast_to(ic, ia_ref.shape)


def _sc_body(t_hbm, ia_hbm, ib_hbm, ic_hbm, out_hbm,
             ia_v, ib_v, ic_v, acc_v, t_sh, sem_i, sem_g, sem_o):
    cid = lax.axis_index("c")
    sid = lax.axis_index("s")
    wid = sid * NC + cid

    @pl.when(sid == 0)
    def _():
        pltpu.sync_copy(t_hbm, t_sh)

    plsc.subcore_barrier()

    base = wid * RPW
    for ch in range(NCH):
        r0 = base + ch * CH
        ca = pltpu.async_copy(ia_hbm.at[pl.ds(r0, CH)], ia_v, sem_i)
        cb = pltpu.async_copy(ib_hbm.at[pl.ds(r0, CH)], ib_v, sem_i)
        cc = pltpu.async_copy(ic_hbm.at[pl.ds(r0, CH)], ic_v, sem_i)
        ca.wait()
        cb.wait()
        cc.wait()
        pltpu.async_copy(t_sh.at[ia_v], acc_v, sem_g).wait()
        gb = pltpu.async_copy(t_sh.at[ib_v], acc_v, sem_g, add=True)
        gc = pltpu.async_copy(t_sh.at[ic_v], acc_v, sem_g, add=True)
        gb.wait()
        gc.wait()
        pltpu.async_copy(acc_v, out_hbm.at[pl.ds(r0, CH)], sem_o).wait()


_sc_call = pl.kernel(
    _sc_body,
    out_type=jax.ShapeDtypeStruct((ROWS, WIDTH), jnp.float32),
    mesh=plsc.VectorSubcoreMesh(core_axis_name="c", subcore_axis_name="s"),
    scratch_types=[
        pltpu.VMEM((CH,), jnp.int32),
        pltpu.VMEM((CH,), jnp.int32),
        pltpu.VMEM((CH,), jnp.int32),
        pltpu.VMEM((CH, WIDTH), jnp.float32),
        pltpu.VMEM_SHARED((NROWS_PAD, WIDTH), jnp.float32),
        pltpu.SemaphoreType.DMA,
        pltpu.SemaphoreType.DMA,
        pltpu.SemaphoreType.DMA,
    ],
)


def kernel(boards, w_quads, w_count):
    batch = boards.shape[0]
    boards_flat = boards.reshape(batch, 36)
    w_cat = jnp.concatenate([w_quads, w_count], axis=0)

    table = pl.pallas_call(
        _table_body,
        out_shape=jax.ShapeDtypeStruct((NROWS_PAD, WIDTH), jnp.float32),
    )(jnp.asarray(_M), w_cat)

    blk = 2048
    grid = batch // blk
    ia, ib, ic = pl.pallas_call(
        _idx_body,
        grid=(grid,),
        in_specs=[
            pl.BlockSpec((blk, 36), lambda i: (i, 0)),
            pl.BlockSpec((36, 8), lambda i: (0, 0)),
        ],
        out_specs=[
            pl.BlockSpec((blk, 4), lambda i: (i, 0)),
            pl.BlockSpec((blk, 4), lambda i: (i, 0)),
            pl.BlockSpec((blk, 4), lambda i: (i, 0)),
        ],
        out_shape=[
            jax.ShapeDtypeStruct((batch, 4), jnp.int32),
            jax.ShapeDtypeStruct((batch, 4), jnp.int32),
            jax.ShapeDtypeStruct((batch, 4), jnp.int32),
        ],
    )(boards_flat, jnp.asarray(_WIDX))

    out = _sc_call(table, ia.reshape(-1), ib.reshape(-1), ic.reshape(-1))
    return out.reshape(batch, 4, WIDTH)


# speedup vs baseline: 1.0507x; 1.0507x over previous
import jax
import jax.numpy as jnp
import numpy as np
from jax import lax
from jax.experimental import pallas as pl
from jax.experimental.pallas import tpu as pltpu
from jax.experimental.pallas import tpu_sc as plsc

BATCH = 16384
WIDTH = 128
NROWS_PAD = 1320
ROWS = BATCH * 4
NC, NS = 2, 16
NW = NC * NS
RPW = ROWS // NW
CH = 128
NCH = RPW // CH

_QSRC = (0, 2, 3, 1)


def _static_m():
    sig = np.zeros((4, 9), dtype=np.int64)
    for p in range(9):
        x, y = divmod(p, 3)
        sig[0, p] = p
        sig[1, p] = 3 * (2 - y) + x
        sig[2, p] = 8 - p
        sig[3, p] = 3 * y + (2 - x)
    inv = np.zeros_like(sig)
    for j in range(4):
        inv[j, sig[j]] = np.arange(9)

    m = np.zeros((NROWS_PAD, 46), np.float32)
    for j in range(4):
        base = 324 * j
        for ia in range(81):
            v = ia
            for t in range(4):
                d = v % 3
                v //= 3
                m[base + ia, 3 * inv[j][t] + d] += 1
        for ib in range(243):
            v = ib
            for t in range(5):
                d = v % 3
                v //= 3
                m[base + 81 + ib, 3 * inv[j][4 + t] + d] += 1
    for c in range(19):
        m[1296 + c, 27 + c] = 1
    return m


_M = _static_m()


def _table_body(m_ref, w_ref, t_ref):
    t_ref[...] = jnp.dot(m_ref[...], w_ref[...],
                         preferred_element_type=jnp.float32)


def _idx_body(bt_ref, out_ref):
    bt = bt_ref[...]
    nzc = jnp.sum((bt != 0).astype(jnp.int32), axis=0, keepdims=True)
    ic = jnp.where(nzc < 19, 1296 + nzc, 1315)
    for j in range(4):
        q9 = 9 * _QSRC[j]
        ia = bt[q9:q9 + 1, :]
        for t in range(1, 4):
            ia = ia + (3 ** t) * bt[q9 + t:q9 + t + 1, :]
        ib = bt[q9 + 4:q9 + 5, :]
        for t in range(1, 5):
            ib = ib + (3 ** t) * bt[q9 + 4 + t:q9 + 5 + t, :]
        out_ref[j:j + 1, :] = ia + 324 * j
        out_ref[4 + j:5 + j, :] = ib + (324 * j + 81)
        out_ref[8 + j:9 + j, :] = ic


def _sc_body(t_hbm, ia_hbm, ib_hbm, ic_hbm, out_hbm,
             ia_v, ib_v, ic_v, acc_v, t_sh,
             sem_i, sem_a, sem_bc, sem_o):
    cid = lax.axis_index("c")
    sid = lax.axis_index("s")
    wid = sid * NC + cid
    base = wid * RPW

    @pl.when(sid == 0)
    def _():
        pltpu.sync_copy(t_hbm, t_sh)

    plsc.subcore_barrier()

    def issue_i(ch):
        r0 = base + ch * CH
        return (
            pltpu.async_copy(ia_hbm.at[pl.ds(r0, CH)], ia_v.at[ch & 1], sem_i),
            pltpu.async_copy(ib_hbm.at[pl.ds(r0, CH)], ib_v.at[ch & 1], sem_i),
            pltpu.async_copy(ic_hbm.at[pl.ds(r0, CH)], ic_v.at[ch & 1], sem_i),
        )

    def issue_a(ch):
        return pltpu.async_copy(t_sh.at[ia_v.at[ch & 1]],
                                acc_v.at[ch % 3], sem_a)

    def issue_bc(ch):
        return (
            pltpu.async_copy(t_sh.at[ib_v.at[ch & 1]],
                             acc_v.at[ch % 3], sem_bc, add=True),
            pltpu.async_copy(t_sh.at[ic_v.at[ch & 1]],
                             acc_v.at[ch % 3], sem_bc, add=True),
        )

    def issue_o(ch):
        return pltpu.async_copy(acc_v.at[ch % 3],
                                out_hbm.at[pl.ds(base + ch * CH, CH)], sem_o)

    di = {0: issue_i(0)}
    for c in di.pop(0):
        c.wait()
    da = {0: issue_a(0)}
    dbc = {}
    do = {}
    for ch in range(NCH):
        if ch + 1 < NCH:
            di[ch + 1] = issue_i(ch + 1)
        da.pop(ch).wait()
        dbc[ch] = issue_bc(ch)
        if ch + 1 < NCH:
            for c in di.pop(ch + 1):
                c.wait()
            if ch >= 2:
                do.pop(ch - 2).wait()
            da[ch + 1] = issue_a(ch + 1)
        for c in dbc.pop(ch):
            c.wait()
        do[ch] = issue_o(ch)
    for ch in sorted(do):
        do[ch].wait()


_sc_call = pl.kernel(
    _sc_body,
    out_type=jax.ShapeDtypeStruct((ROWS, WIDTH), jnp.float32),
    mesh=plsc.VectorSubcoreMesh(core_axis_name="c", subcore_axis_name="s"),
    scratch_types=[
        pltpu.VMEM((2, CH), jnp.int32),
        pltpu.VMEM((2, CH), jnp.int32),
        pltpu.VMEM((2, CH), jnp.int32),
        pltpu.VMEM((3, CH, WIDTH), jnp.float32),
        pltpu.VMEM_SHARED((NROWS_PAD, WIDTH), jnp.float32),
        pltpu.SemaphoreType.DMA,
        pltpu.SemaphoreType.DMA,
        pltpu.SemaphoreType.DMA,
        pltpu.SemaphoreType.DMA,
    ],
)


def kernel(boards, w_quads, w_count):
    batch = boards.shape[0]
    boards_t = boards.reshape(batch, 36).T
    w_cat = jnp.concatenate([w_quads, w_count], axis=0)

    table = pl.pallas_call(
        _table_body,
        out_shape=jax.ShapeDtypeStruct((NROWS_PAD, WIDTH), jnp.float32),
    )(jnp.asarray(_M), w_cat)

    blkc = 4096
    idx_all = pl.pallas_call(
        _idx_body,
        grid=(batch // blkc,),
        in_specs=[pl.BlockSpec((36, blkc), lambda i: (0, i))],
        out_specs=pl.BlockSpec((12, blkc), lambda i: (0, i)),
        out_shape=jax.ShapeDtypeStruct((12, batch), jnp.int32),
    )(boards_t)

    ia = idx_all[0:4, :].T.reshape(-1)
    ib = idx_all[4:8, :].T.reshape(-1)
    ic = idx_all[8:12, :].T.reshape(-1)
    out = _sc_call(table, ia, ib, ic)
    return out.reshape(batch, 4, WIDTH)
